# Initial kernel scaffold; baseline (speedup 1.0000x reference)
import jax
import jax.numpy as jnp
from jax.experimental import pallas as pl


def kernel(x, W1_w, W1_b, W2_w, W2_b, phi_w, phi_i, phi_j, h1_w, h1_b, h2_w, h2_b, S_m_x, S_m_y):
    raise NotImplementedError("write your pallas kernel here")


# speedup vs baseline: 28.0687x; 28.0687x over previous
import functools

import jax
import jax.numpy as jnp
from jax import lax
from jax.experimental import pallas as pl
from jax.experimental.pallas import tpu as pltpu
from jax.experimental.pallas import tpu_sc as plsc

_N = 4096
_K = 32
_H = 64
_WROWS = 16
_WCAND = _WROWS * _H
_C = 128
_RADIUS = 0.2
_NKNOTS = 32


def _nbr_body(lin_smem, knots_smem, smx_smem, smy_smem,
              lin_col, phiw, phii, phij,
              jout, prod_out, psis_out, phis_out):
    r0 = pl.program_id(0)
    start = jnp.clip(r0 - 7, 0, _H - _WROWS)
    lin = lin_col[...]
    riota = lax.broadcasted_iota(jnp.int32, (_H, _WCAND), 0)
    liota = lax.broadcasted_iota(jnp.int32, (_H, _WCAND), 1)
    lin_b = jnp.broadcast_to(lin, (_H, _WCAND))
    yc = jnp.sum(jnp.where(liota % _H == riota, lin_b, 0.0), axis=0, keepdims=True)
    xc = jnp.sum(jnp.where(liota // _H + start == riota, lin_b, 0.0), axis=0,
                 keepdims=True)
    xi = lin_smem[r0]
    dx = xi - xc
    dy = lin - yc
    dist = jnp.sqrt(dx * dx + dy * dy)
    dist = jnp.where(dist <= _RADIUS, dist, jnp.inf)
    sel = []
    for _ in range(_K):
        m = jnp.min(dist, axis=1, keepdims=True)
        cand = jnp.where(dist == m, liota, jnp.int32(1 << 30))
        amin = jnp.min(cand, axis=1, keepdims=True)
        sel.append(amin)
        dist = jnp.where(liota == amin, jnp.inf, dist)
    lsel = jnp.concatenate(sel, axis=1)
    jout[0] = start * _H + lsel
    rsel = lsel // _H
    csel = lsel % _H
    cjx = jnp.zeros((_H, _K), jnp.float32)
    cjy = jnp.zeros((_H, _K), jnp.float32)
    for t in range(_WROWS):
        cjx = jnp.where(rsel == t, lin_smem[start + t], cjx)
    for c in range(_H):
        cjy = jnp.where(csel == c, lin_smem[c], cjy)
    relx = xi - cjx
    rely = lin - cjy
    psi_x = jnp.zeros((_H, _K), jnp.float32)
    psi_y = jnp.zeros((_H, _K), jnp.float32)
    for t in range(_NKNOTS):
        kv = knots_smem[t]
        psi_x = psi_x + jnp.maximum(1.0 - jnp.abs(relx - kv), 0.0) * smx_smem[t]
        psi_y = psi_y + jnp.maximum(1.0 - jnp.abs(rely - kv), 0.0) * smy_smem[t]
    psi = psi_x * psi_y
    pw = phiw[...]
    pi_ = phii[...]
    pj_ = phij[...]
    p0 = pw * pi_[:, 0:1]
    p1 = pw * pi_[:, 1:2]
    m00 = jnp.sum(p0 * pj_[:, 0:1])
    m01 = jnp.sum(p0 * pj_[:, 1:2])
    m10 = jnp.sum(p1 * pj_[:, 0:1])
    m11 = jnp.sum(p1 * pj_[:, 1:2])
    phiv = (m00 * cjx + m01 * cjy) * xi + (m10 * cjx + m11 * cjy) * lin
    prod_out[0] = psi * phiv
    psis_out[0] = jnp.sum(jnp.abs(psi)).reshape(1, 1)
    phis_out[0] = jnp.sum(jnp.abs(phiv)).reshape(1, 1)


def _neighbors_and_weights(lin, knots, S_m_x, S_m_y, phi_w, phi_i, phi_j):
    out_shapes = [
        jax.ShapeDtypeStruct((_H, _H, _K), jnp.int32),
        jax.ShapeDtypeStruct((_H, _H, _K), jnp.float32),
        jax.ShapeDtypeStruct((_H, 1, 1), jnp.float32),
        jax.ShapeDtypeStruct((_H, 1, 1), jnp.float32),
    ]
    return pl.pallas_call(
        _nbr_body,
        grid=(_H,),
        in_specs=[
            pl.BlockSpec(memory_space=pltpu.SMEM),
            pl.BlockSpec(memory_space=pltpu.SMEM),
            pl.BlockSpec(memory_space=pltpu.SMEM),
            pl.BlockSpec(memory_space=pltpu.SMEM),
            pl.BlockSpec((_H, 1), lambda r: (0, 0)),
            pl.BlockSpec((_C, 1), lambda r: (0, 0)),
            pl.BlockSpec((_C, 2), lambda r: (0, 0)),
            pl.BlockSpec((_C, 2), lambda r: (0, 0)),
        ],
        out_specs=[
            pl.BlockSpec((1, _H, _K), lambda r: (r, 0, 0)),
            pl.BlockSpec((1, _H, _K), lambda r: (r, 0, 0)),
            pl.BlockSpec((1, 1, 1), lambda r: (r, 0, 0)),
            pl.BlockSpec((1, 1, 1), lambda r: (r, 0, 0)),
        ],
        out_shape=out_shapes,
    )(lin, knots, S_m_x, S_m_y, lin.reshape(_H, 1),
      phi_w.reshape(_C, 1), phi_i, phi_j)


_G = 16
_NW = 32
_DPW = (2 * _N) // _NW
_NCH = _DPW // _G


def _agg_body(x_hbm, gidx_hbm, w_hbm, out_hbm, idx_v, rows_v, w_v, out_v, sem):
    cid = lax.axis_index("c")
    sid = lax.axis_index("s")
    wid = sid * 2 + cid
    pltpu.sync_copy(gidx_hbm.at[pl.ds(wid * (_DPW * _K // 128), _DPW * _K // 128), :],
                    idx_v)
    pltpu.sync_copy(w_hbm.at[pl.ds(wid * _DPW, _DPW), :], w_v)

    def chunk(c, carry):
        d0 = wid * _DPW + c * _G
        handles = [
            pltpu.async_copy(x_hbm.at[idx_v.at[c * 4 + j]],
                             rows_v.at[pl.ds(j * 128, 128), :], sem)
            for j in range(4)
        ]
        for h in handles:
            h.wait()

        def g_body(g, carry2):
            accs = [jnp.zeros((16,), jnp.float32) for _ in range(8)]
            wrow = c * _G + g
            whalf = (w_v[wrow, pl.ds(0, 16)], w_v[wrow, pl.ds(16, 16)])
            for k in range(_K):
                wb = lax.gather(
                    whalf[k // 16],
                    jnp.full((16, 1), k % 16, jnp.int32),
                    lax.GatherDimensionNumbers(
                        offset_dims=(), collapsed_slice_dims=(0,),
                        start_index_map=(0,)),
                    (1,),
                    mode=lax.GatherScatterMode.PROMISE_IN_BOUNDS)
                row = g * _K + k
                for c8 in range(8):
                    accs[c8] = accs[c8] + wb * rows_v[row, pl.ds(c8 * 16, 16)]
            for c8 in range(8):
                out_v[g, pl.ds(c8 * 16, 16)] = accs[c8]
            return carry2

        lax.fori_loop(0, _G, g_body, 0)
        pltpu.sync_copy(out_v, out_hbm.at[pl.ds(d0, _G), :])
        return carry

    lax.fori_loop(0, _NCH, chunk, 0)


def _aggregate(x_flat, gidx2d, w2):
    mesh = plsc.VectorSubcoreMesh(core_axis_name="c", subcore_axis_name="s")
    kern = functools.partial(
        pl.kernel,
        mesh=mesh,
        out_type=jax.ShapeDtypeStruct((2 * _N, _C), jnp.float32),
        scratch_types=[
            pltpu.VMEM((_DPW * _K // 128, 128), jnp.int32),
            pltpu.VMEM((_G * _K, _C), jnp.float32),
            pltpu.VMEM((_DPW, _K), jnp.float32),
            pltpu.VMEM((_G, _C), jnp.float32),
            pltpu.SemaphoreType.DMA,
        ],
    )(_agg_body)
    return kern(x_flat, gidx2d, w2)


def _mlp_body(scale_smem, x_ref, w1t_ref, b1_ref, w2t_ref, b2_ref, agg_ref,
              o_ref):
    h = jnp.dot(x_ref[...], w1t_ref[...], preferred_element_type=jnp.float32)
    h = jnp.maximum(h + b1_ref[...], 0.0)
    o = jnp.dot(h, w2t_ref[...], preferred_element_type=jnp.float32)
    o_ref[...] = o + b2_ref[...] + scale_smem[0] * agg_ref[...]


def _mlp_combine(x_flat, W1T, W1_b, W2T, W2_b, agg, scale):
    rows = 2 * _N
    br = 512
    return pl.pallas_call(
        _mlp_body,
        grid=(rows // br,),
        in_specs=[
            pl.BlockSpec(memory_space=pltpu.SMEM),
            pl.BlockSpec((br, _C), lambda r: (r, 0)),
            pl.BlockSpec((_C, 2 * _C), lambda r: (0, 0)),
            pl.BlockSpec((1, 2 * _C), lambda r: (0, 0)),
            pl.BlockSpec((2 * _C, _C), lambda r: (0, 0)),
            pl.BlockSpec((1, _C), lambda r: (0, 0)),
            pl.BlockSpec((br, _C), lambda r: (r, 0)),
        ],
        out_specs=pl.BlockSpec((br, _C), lambda r: (r, 0)),
        out_shape=jax.ShapeDtypeStruct((rows, _C), jnp.float32),
    )(scale, x_flat, W1T, W1_b.reshape(1, 2 * _C), W2T, W2_b.reshape(1, _C),
      agg)


def kernel(x, W1_w, W1_b, W2_w, W2_b, phi_w, phi_i, phi_j,
           h1_w, h1_b, h2_w, h2_b, S_m_x, S_m_y):
    lin = jnp.linspace(0.0, 1.0, _H).astype(jnp.float32)
    knots = jnp.linspace(0.0, 1.0, _NKNOTS).astype(jnp.float32)
    jout, prod, psis, phis = _neighbors_and_weights(
        lin, knots, S_m_x, S_m_y, phi_w, phi_i, phi_j)
    jflat = jout.reshape(_N, _K)
    prod_flat = prod.reshape(_N, _K)
    mpsi = jnp.sum(psis) / (_N * _K)
    mphi = jnp.sum(phis) / (_N * _K)
    scale = 1.0 / ((mpsi + 1e-6) * (mphi + 1e-6) * jnp.float32(_K))
    gidx = jnp.concatenate([jflat, jflat + _N], axis=0)
    gidx = gidx.reshape((2 * _N * _K) // 128, 128)
    w2 = jnp.concatenate([prod_flat, prod_flat], axis=0)
    x_flat = x.reshape(2 * _N, _C)
    agg = _aggregate(x_flat, gidx, w2)
    out = _mlp_combine(x_flat, W1_w.T, W1_b, W2_w.T, W2_b, agg,
                       scale.reshape(1))
    return out.reshape(2, _N, _C)


# speedup vs baseline: 28.7331x; 1.0237x over previous
import functools

import jax
import jax.numpy as jnp
from jax import lax
from jax.experimental import pallas as pl
from jax.experimental.pallas import tpu as pltpu
from jax.experimental.pallas import tpu_sc as plsc

_N = 4096
_K = 32
_H = 64
_WROWS = 9
_WCAND = _WROWS * _H
_C = 128
_NKNOTS = 32


def _nbr_body(lin_smem, knots_smem, smx_smem, smy_smem,
              lin_col, phiw, phii, phij,
              jout, prod_out, psis_out, phis_out):
    r0 = pl.program_id(0)
    start = jnp.clip(r0 - 4, 0, _H - _WROWS)
    lin = lin_col[...]
    riota = lax.broadcasted_iota(jnp.int32, (_H, _WCAND), 0)
    liota = lax.broadcasted_iota(jnp.int32, (_H, _WCAND), 1)
    lin_b = jnp.broadcast_to(lin, (_H, _WCAND))
    yc = jnp.sum(jnp.where(liota % _H == riota, lin_b, 0.0), axis=0,
                 keepdims=True)
    xc = jnp.sum(jnp.where(liota // _H + start == riota, lin_b, 0.0), axis=0,
                 keepdims=True)
    xi = lin_smem[r0]
    dx = xi - xc
    dy = lin - yc
    dist = jnp.sqrt(dx * dx + dy * dy)
    sel = []
    for _ in range(_K):
        m = jnp.min(dist, axis=1, keepdims=True)
        cand = jnp.where(dist == m, liota, jnp.int32(1 << 30))
        amin = jnp.min(cand, axis=1, keepdims=True)
        sel.append(amin)
        dist = jnp.where(cand == amin, jnp.inf, dist)
    lsel = jnp.concatenate(sel, axis=1)
    jout[0] = start * _H + lsel
    rsel = lsel // _H
    csel = lsel % _H
    cjx = jnp.zeros((_H, _K), jnp.float32)
    cjy = jnp.zeros((_H, _K), jnp.float32)
    for t in range(_WROWS):
        cjx = jnp.where(rsel == t, lin_smem[start + t], cjx)
    for c in range(_H):
        cjy = jnp.where(csel == c, lin_smem[c], cjy)
    relx = xi - cjx
    rely = lin - cjy
    psi_x = jnp.zeros((_H, _K), jnp.float32)
    psi_y = jnp.zeros((_H, _K), jnp.float32)
    for t in range(_NKNOTS):
        kv = knots_smem[t]
        psi_x = psi_x + jnp.maximum(1.0 - jnp.abs(relx - kv), 0.0) * smx_smem[t]
        psi_y = psi_y + jnp.maximum(1.0 - jnp.abs(rely - kv), 0.0) * smy_smem[t]
    psi = psi_x * psi_y
    pw = phiw[...]
    pi_ = phii[...]
    pj_ = phij[...]
    p0 = pw * pi_[:, 0:1]
    p1 = pw * pi_[:, 1:2]
    m00 = jnp.sum(p0 * pj_[:, 0:1])
    m01 = jnp.sum(p0 * pj_[:, 1:2])
    m10 = jnp.sum(p1 * pj_[:, 0:1])
    m11 = jnp.sum(p1 * pj_[:, 1:2])
    phiv = (m00 * cjx + m01 * cjy) * xi + (m10 * cjx + m11 * cjy) * lin
    prod_out[0] = psi * phiv
    psis_out[0] = jnp.sum(jnp.abs(psi)).reshape(1, 1)
    phis_out[0] = jnp.sum(jnp.abs(phiv)).reshape(1, 1)


def _neighbors_and_weights(lin, knots, S_m_x, S_m_y, phi_w, phi_i, phi_j):
    out_shapes = [
        jax.ShapeDtypeStruct((_H, _H, _K), jnp.int32),
        jax.ShapeDtypeStruct((_H, _H, _K), jnp.float32),
        jax.ShapeDtypeStruct((_H, 1, 1), jnp.float32),
        jax.ShapeDtypeStruct((_H, 1, 1), jnp.float32),
    ]
    return pl.pallas_call(
        _nbr_body,
        grid=(_H,),
        in_specs=[
            pl.BlockSpec(memory_space=pltpu.SMEM),
            pl.BlockSpec(memory_space=pltpu.SMEM),
            pl.BlockSpec(memory_space=pltpu.SMEM),
            pl.BlockSpec(memory_space=pltpu.SMEM),
            pl.BlockSpec((_H, 1), lambda r: (0, 0)),
            pl.BlockSpec((_C, 1), lambda r: (0, 0)),
            pl.BlockSpec((_C, 2), lambda r: (0, 0)),
            pl.BlockSpec((_C, 2), lambda r: (0, 0)),
        ],
        out_specs=[
            pl.BlockSpec((1, _H, _K), lambda r: (r, 0, 0)),
            pl.BlockSpec((1, _H, _K), lambda r: (r, 0, 0)),
            pl.BlockSpec((1, 1, 1), lambda r: (r, 0, 0)),
            pl.BlockSpec((1, 1, 1), lambda r: (r, 0, 0)),
        ],
        out_shape=out_shapes,
    )(lin, knots, S_m_x, S_m_y, lin.reshape(_H, 1),
      phi_w.reshape(_C, 1), phi_i, phi_j)


_G = 16
_NW = 32
_DPW = (2 * _N) // _NW
_NCH = _DPW // _G


def _agg_body(x_hbm, gidx_hbm, w_hbm, out_hbm, idx_v, rows_v, w_v, out_v, sem):
    cid = lax.axis_index("c")
    sid = lax.axis_index("s")
    wid = sid * 2 + cid
    pltpu.sync_copy(gidx_hbm.at[pl.ds(wid * (_DPW * _K // 128), _DPW * _K // 128), :],
                    idx_v)
    pltpu.sync_copy(w_hbm.at[pl.ds(wid * _DPW, _DPW), :], w_v)

    def chunk(c, carry):
        d0 = wid * _DPW + c * _G
        handles = [
            pltpu.async_copy(x_hbm.at[idx_v.at[c * 4 + j]],
                             rows_v.at[pl.ds(j * 128, 128), :], sem)
            for j in range(4)
        ]
        for h in handles:
            h.wait()

        def g_body(g, carry2):
            accs = [jnp.zeros((16,), jnp.float32) for _ in range(8)]
            wrow = c * _G + g
            whalf = (w_v[wrow, pl.ds(0, 16)], w_v[wrow, pl.ds(16, 16)])
            for k in range(_K):
                wb = lax.gather(
                    whalf[k // 16],
                    jnp.full((16, 1), k % 16, jnp.int32),
                    lax.GatherDimensionNumbers(
                        offset_dims=(), collapsed_slice_dims=(0,),
                        start_index_map=(0,)),
                    (1,),
                    mode=lax.GatherScatterMode.PROMISE_IN_BOUNDS)
                row = g * _K + k
                for c8 in range(8):
                    accs[c8] = accs[c8] + wb * rows_v[row, pl.ds(c8 * 16, 16)]
            for c8 in range(8):
                out_v[g, pl.ds(c8 * 16, 16)] = accs[c8]
            return carry2

        lax.fori_loop(0, _G, g_body, 0)
        pltpu.sync_copy(out_v, out_hbm.at[pl.ds(d0, _G), :])
        return carry

    lax.fori_loop(0, _NCH, chunk, 0)


def _aggregate(x_flat, gidx2d, w2):
    mesh = plsc.VectorSubcoreMesh(core_axis_name="c", subcore_axis_name="s")
    kern = functools.partial(
        pl.kernel,
        mesh=mesh,
        out_type=jax.ShapeDtypeStruct((2 * _N, _C), jnp.float32),
        scratch_types=[
            pltpu.VMEM((_DPW * _K // 128, 128), jnp.int32),
            pltpu.VMEM((_G * _K, _C), jnp.float32),
            pltpu.VMEM((_DPW, _K), jnp.float32),
            pltpu.VMEM((_G, _C), jnp.float32),
            pltpu.SemaphoreType.DMA,
        ],
    )(_agg_body)
    return kern(x_flat, gidx2d, w2)


def _mlp_body(scale_smem, x_ref, w1t_ref, b1_ref, w2t_ref, b2_ref, agg_ref,
              o_ref):
    h = jnp.dot(x_ref[...], w1t_ref[...], preferred_element_type=jnp.float32)
    h = jnp.maximum(h + b1_ref[...], 0.0)
    o = jnp.dot(h, w2t_ref[...], preferred_element_type=jnp.float32)
    o_ref[...] = o + b2_ref[...] + scale_smem[0] * agg_ref[...]


def _mlp_combine(x_flat, W1T, W1_b, W2T, W2_b, agg, scale):
    rows = 2 * _N
    br = 512
    return pl.pallas_call(
        _mlp_body,
        grid=(rows // br,),
        in_specs=[
            pl.BlockSpec(memory_space=pltpu.SMEM),
            pl.BlockSpec((br, _C), lambda r: (r, 0)),
            pl.BlockSpec((_C, 2 * _C), lambda r: (0, 0)),
            pl.BlockSpec((1, 2 * _C), lambda r: (0, 0)),
            pl.BlockSpec((2 * _C, _C), lambda r: (0, 0)),
            pl.BlockSpec((1, _C), lambda r: (0, 0)),
            pl.BlockSpec((br, _C), lambda r: (r, 0)),
        ],
        out_specs=pl.BlockSpec((br, _C), lambda r: (r, 0)),
        out_shape=jax.ShapeDtypeStruct((rows, _C), jnp.float32),
    )(scale, x_flat, W1T, W1_b.reshape(1, 2 * _C), W2T, W2_b.reshape(1, _C),
      agg)


def kernel(x, W1_w, W1_b, W2_w, W2_b, phi_w, phi_i, phi_j,
           h1_w, h1_b, h2_w, h2_b, S_m_x, S_m_y):
    lin = jnp.linspace(0.0, 1.0, _H).astype(jnp.float32)
    knots = jnp.linspace(0.0, 1.0, _NKNOTS).astype(jnp.float32)
    jout, prod, psis, phis = _neighbors_and_weights(
        lin, knots, S_m_x, S_m_y, phi_w, phi_i, phi_j)
    jflat = jout.reshape(_N, _K)
    prod_flat = prod.reshape(_N, _K)
    mpsi = jnp.sum(psis) / (_N * _K)
    mphi = jnp.sum(phis) / (_N * _K)
    scale = 1.0 / ((mpsi + 1e-6) * (mphi + 1e-6) * jnp.float32(_K))
    gidx = jnp.concatenate([jflat, jflat + _N], axis=0)
    gidx = gidx.reshape((2 * _N * _K) // 128, 128)
    w2 = jnp.concatenate([prod_flat, prod_flat], axis=0)
    x_flat = x.reshape(2 * _N, _C)
    agg = _aggregate(x_flat, gidx, w2)
    out = _mlp_combine(x_flat, W1_w.T, W1_b, W2_w.T, W2_b, agg,
                       scale.reshape(1))
    return out.reshape(2, _N, _C)
